# baseline (speedup 1.0000x reference)
import functools

import jax
import jax.numpy as jnp
from jax import lax
from jax.experimental import pallas as pl
from jax.experimental.pallas import tpu as pltpu
from jax.experimental.pallas import tpu_sc as plsc

B = 16384
V = 1000
H = 128
NF = 6
NC = 2
NS = 16
NW = NC * NS
B_SC = 12288
B_TC = B - B_SC
VP = 1024
BPW = B_SC // NW
CH = 128
NCHUNK = BPW // CH
LANES = 16
KS = H // LANES


_GDN = lax.GatherDimensionNumbers(
    offset_dims=(), collapsed_slice_dims=(0,), start_index_map=(0,))


def _lane_perm(x, idx):
    return lax.gather(x, idx[:, None], _GDN, slice_sizes=(1,),
                      mode=lax.GatherScatterMode.PROMISE_IN_BOUNDS)


def _mm_body(emb_ref, w1_ref, b1_ref, out_ref):
    out_ref[0] = (jnp.dot(emb_ref[0], w1_ref[...],
                          preferred_element_type=jnp.float32)
                  + b1_ref[...] * (1.0 / NF))


def _precompute_m(embs, w1, b1):
    return pl.pallas_call(
        _mm_body,
        grid=(NF,),
        in_specs=[
            pl.BlockSpec((1, V, H), lambda f: (f, 0, 0)),
            pl.BlockSpec((H, H), lambda f: (f, 0)),
            pl.BlockSpec((H,), lambda f: (0,)),
        ],
        out_specs=pl.BlockSpec((1, V, H), lambda f: (f, 0, 0)),
        out_shape=jax.ShapeDtypeStruct((NF, V, H), jnp.float32),
    )(embs, w1, b1)


_mesh = plsc.VectorSubcoreMesh(core_axis_name="c", subcore_axis_name="s")


@functools.partial(
    pl.kernel,
    out_type=jax.ShapeDtypeStruct((B_SC,), jnp.float32),
    mesh=_mesh,
    scratch_types=(
        [pltpu.VMEM((NCHUNK, CH), jnp.int32) for _ in range(NF)]
        + [pltpu.VMEM((CH, H), jnp.float32) for _ in range(NCHUNK)]
        + [
            pltpu.VMEM((H,), jnp.float32),
            pltpu.VMEM((LANES,), jnp.float32),
            pltpu.VMEM((BPW,), jnp.float32),
        ]
        + [pltpu.SemaphoreType.DMA for _ in range(NCHUNK)]
    ),
)
def _sc_fused(i0, i1, i2, i3, i4, i5,
              m0, m1, m2, m3, m4, m5,
              w2_hbm, b2_hbm,
              out_hbm,
              x0, x1, x2, x3, x4, x5,
              acc_a, acc_b, acc_c, w2_v, b2_v, out_v,
              sem_a, sem_b, sem_c):
    idx_hbm = [i0, i1, i2, i3, i4, i5]
    m_hbm = [m0, m1, m2, m3, m4, m5]
    xv = [x0, x1, x2, x3, x4, x5]

    wid = lax.axis_index("s") * NC + lax.axis_index("c")

    for f in range(NF):
        pltpu.sync_copy(idx_hbm[f].at[wid], xv[f])
    pltpu.sync_copy(w2_hbm, w2_v)
    pltpu.sync_copy(b2_hbm, b2_v)

    w2k = [w2_v[pl.ds(k * LANES, LANES)] for k in range(KS)]
    b2vec = b2_v[...]
    lane = lax.iota(jnp.int32, LANES)
    zvec = jnp.zeros((LANES,), jnp.float32)
    xor_tabs = [jnp.bitwise_xor(lane, s) for s in (8, 4, 2, 1)]

    bufs = [acc_a, acc_b, acc_c]
    sems = [sem_a, sem_b, sem_c]

    def fire(c, buf, sem):
        def zero_body(r, carry2):
            for k in range(KS):
                buf[r, pl.ds(k * LANES, LANES)] = zvec
            return carry2

        lax.fori_loop(0, CH, zero_body, 0)
        return [pltpu.async_copy(m_hbm[f].at[xv[f].at[c]], buf, sem,
                                 add=True)
                for f in range(NF)]

    def compute(c, buf):
        def group_body(g, carry2):
            y = zvec
            for r16 in range(LANES):
                r = g * LANES + r16
                p = zvec
                for k in range(KS):
                    h = jnp.maximum(buf[r, pl.ds(k * LANES, LANES)], 0.0)
                    p = p + h * w2k[k]
                for t in xor_tabs:
                    p = p + _lane_perm(p, t)
                y = jnp.where(lane == r16, p, y)
            z = y + b2vec
            s = 1.0 / (1.0 + jnp.exp(-z))
            out_v[pl.ds(c * CH + g * LANES, LANES)] = s
            return carry2

        lax.fori_loop(0, CH // LANES, group_body, 0)

    pend = [fire(c, bufs[c], sems[c]) for c in range(NCHUNK)]
    for c in range(NCHUNK):
        for cp in pend[c]:
            cp.wait()
        compute(c, bufs[c])

    pltpu.sync_copy(out_v, out_hbm.at[pl.ds(wid * BPW, BPW)])


_TBLK = 512


def _onehot_body(c0, c1, c2, c3, c4, c5, mcat_ref, w2_ref, b2_ref, o_ref):
    cs = [c0, c1, c2, c3, c4, c5]
    iota = lax.broadcasted_iota(jnp.int32, (_TBLK, NF * VP), 1)
    x = None
    for f in range(NF):
        eq = (iota == cs[f][...][:, None]).astype(jnp.bfloat16)
        x = eq if x is None else x + eq
    h = jnp.dot(x, mcat_ref[...], preferred_element_type=jnp.float32)
    z = jnp.sum(jnp.maximum(h, 0.0) * w2_ref[...][None, :],
                axis=1) + b2_ref[0]
    o_ref[...] = jax.nn.sigmoid(z)


def _tc_onehot(cidxs, mcat, w2, b2):
    return pl.pallas_call(
        _onehot_body,
        grid=(B_TC // _TBLK,),
        in_specs=(
            [pl.BlockSpec((_TBLK,), lambda i: (i,)) for _ in range(NF)]
            + [
                pl.BlockSpec((NF * VP, H), lambda i: (0, 0)),
                pl.BlockSpec((H,), lambda i: (0,)),
                pl.BlockSpec(memory_space=pltpu.SMEM),
            ]
        ),
        out_specs=pl.BlockSpec((_TBLK,), lambda i: (i,)),
        out_shape=jax.ShapeDtypeStruct((B_TC,), jnp.float32),
    )(*cidxs, mcat, w2, b2)


def kernel(deviceid, adid, adsize, adx, bundle, business_type,
           emb0, emb1, emb2, emb3, emb4, emb5, W1, b1, W2, b2):
    raw = [a.astype(jnp.int32)
           for a in (deviceid, adid, adsize, adx, bundle, business_type)]
    idxs = [a[:B_SC].reshape(NW, NCHUNK, CH) for a in raw]
    cidxs = [raw[f][B_SC:] + f * VP for f in range(NF)]
    embs = jnp.stack([emb0, emb1, emb2, emb3, emb4, emb5])
    m = _precompute_m(embs, W1, b1)
    ms = [m[f] for f in range(NF)]
    mcat = jnp.concatenate(
        [jnp.pad(m[f], ((0, VP - V), (0, 0))) for f in range(NF)]
    ).astype(jnp.bfloat16)
    w2 = W2.reshape(H)
    b2f = b2.astype(jnp.float32)
    b2v = jnp.broadcast_to(b2, (LANES,)).astype(jnp.float32)
    out_sc = _sc_fused(*idxs, *ms, w2, b2v)
    out_tc = _tc_onehot(cidxs, mcat, w2, b2f)
    return jnp.concatenate([out_sc, out_tc])


# speedup vs baseline: 1.6379x; 1.6379x over previous
import functools

import jax
import jax.numpy as jnp
from jax import lax
from jax.experimental import pallas as pl
from jax.experimental.pallas import tpu as pltpu
from jax.experimental.pallas import tpu_sc as plsc

B = 16384
V = 1000
H = 128
NF = 6
NC = 2
NS = 16
NW = NC * NS
BPW = B // NW
CH = 128
NCHUNK = BPW // CH
LANES = 16
KS = H // LANES


_GDN = lax.GatherDimensionNumbers(
    offset_dims=(), collapsed_slice_dims=(0,), start_index_map=(0,))


def _lane_perm(x, idx):
    return lax.gather(x, idx[:, None], _GDN, slice_sizes=(1,),
                      mode=lax.GatherScatterMode.PROMISE_IN_BOUNDS)


def _mm_body(emb_ref, w1_ref, b1_ref, out_ref):
    out_ref[0] = (jnp.dot(emb_ref[0], w1_ref[...],
                          preferred_element_type=jnp.float32)
                  + b1_ref[...] * (1.0 / NF))


def _precompute_m(embs, w1, b1):
    return pl.pallas_call(
        _mm_body,
        grid=(NF,),
        in_specs=[
            pl.BlockSpec((1, V, H), lambda f: (f, 0, 0)),
            pl.BlockSpec((H, H), lambda f: (f, 0)),
            pl.BlockSpec((H,), lambda f: (0,)),
        ],
        out_specs=pl.BlockSpec((1, V, H), lambda f: (f, 0, 0)),
        out_shape=jax.ShapeDtypeStruct((NF, V, H), jnp.float32),
    )(embs, w1, b1)


_mesh = plsc.VectorSubcoreMesh(core_axis_name="c", subcore_axis_name="s")


@functools.partial(
    pl.kernel,
    out_type=jax.ShapeDtypeStruct((B,), jnp.float32),
    mesh=_mesh,
    scratch_types=(
        [pltpu.VMEM((NCHUNK, CH), jnp.int32) for _ in range(NF)]
        + [pltpu.VMEM((CH, H), jnp.float32) for _ in range(NCHUNK)]
        + [
            pltpu.VMEM((H,), jnp.float32),
            pltpu.VMEM((LANES,), jnp.float32),
            pltpu.VMEM((BPW,), jnp.float32),
        ]
        + [pltpu.SemaphoreType.DMA for _ in range(NCHUNK)]
    ),
)
def _sc_fused(i0, i1, i2, i3, i4, i5,
              m0, m1, m2, m3, m4, m5,
              w2_hbm, b2_hbm,
              out_hbm,
              x0, x1, x2, x3, x4, x5,
              acc_a, acc_b, acc_c, acc_d, w2_v, b2_v, out_v,
              sem_a, sem_b, sem_c, sem_d):
    idx_hbm = [i0, i1, i2, i3, i4, i5]
    m_hbm = [m0, m1, m2, m3, m4, m5]
    xv = [x0, x1, x2, x3, x4, x5]

    wid = lax.axis_index("s") * NC + lax.axis_index("c")

    row0 = wid * NCHUNK
    for f in range(NF):
        pltpu.sync_copy(idx_hbm[f].at[pl.ds(row0, NCHUNK)], xv[f])
    pltpu.sync_copy(w2_hbm, w2_v)
    pltpu.sync_copy(b2_hbm, b2_v)

    w2k = [w2_v[pl.ds(k * LANES, LANES)] for k in range(KS)]
    b2vec = b2_v[...]
    lane = lax.iota(jnp.int32, LANES)
    zvec = jnp.zeros((LANES,), jnp.float32)
    xor_tabs = [jnp.bitwise_xor(lane, s) for s in (8, 4, 2, 1)]

    bufs = [acc_a, acc_b, acc_c, acc_d]
    sems = [sem_a, sem_b, sem_c, sem_d]

    def fire(c, buf, sem):
        def zero_body(r, carry2):
            for k in range(KS):
                buf[r, pl.ds(k * LANES, LANES)] = zvec
            return carry2

        lax.fori_loop(0, CH, zero_body, 0)
        return [pltpu.async_copy(m_hbm[f].at[xv[f].at[c]], buf, sem,
                                 add=True)
                for f in range(NF)]

    def compute(c, buf):
        def group_body(g, carry2):
            y = zvec
            for r16 in range(LANES):
                r = g * LANES + r16
                p = zvec
                for k in range(KS):
                    h = jnp.maximum(buf[r, pl.ds(k * LANES, LANES)], 0.0)
                    p = p + h * w2k[k]
                for t in xor_tabs:
                    p = p + _lane_perm(p, t)
                y = jnp.where(lane == r16, p, y)
            z = y + b2vec
            s = 1.0 / (1.0 + jnp.exp(-z))
            out_v[pl.ds(c * CH + g * LANES, LANES)] = s
            return carry2

        lax.fori_loop(0, CH // LANES, group_body, 0)

    pend = [fire(c, bufs[c], sems[c]) for c in range(NCHUNK)]
    for c in range(NCHUNK):
        for cp in pend[c]:
            cp.wait()
        compute(c, bufs[c])

    pltpu.sync_copy(out_v, out_hbm.at[pl.ds(wid * BPW, BPW)])


def kernel(deviceid, adid, adsize, adx, bundle, business_type,
           emb0, emb1, emb2, emb3, emb4, emb5, W1, b1, W2, b2):
    idxs = [a.astype(jnp.int32).reshape(B // CH, CH)
            for a in (deviceid, adid, adsize, adx, bundle, business_type)]
    embs = jnp.stack([emb0, emb1, emb2, emb3, emb4, emb5])
    m = _precompute_m(embs, W1, b1)
    ms = [m[f] for f in range(NF)]
    w2 = W2.reshape(H)
    b2v = jnp.broadcast_to(b2, (LANES,)).astype(jnp.float32)
    return _sc_fused(*idxs, *ms, w2, b2v)
